# baseline (speedup 1.0000x reference)
import jax
import jax.numpy as jnp
from jax.experimental import pallas as pl


def _mp(h, src, dst, e, Wself, Wnbr, We, b):
    msg = jnp.take(h, src, axis=0) @ Wnbr + e @ We
    agg = jnp.zeros((h.shape[0], Wnbr.shape[1]), h.dtype).at[dst].add(msg)
    return jax.nn.relu(h @ Wself + agg + b)


def _enc(attr, ei, eattr, params):
    Ws1, Wn1, We1, b1, Ws2, Wn2, We2, b2, Wo, bo = params
    src, dst = ei[0], ei[1]
    h = _mp(attr, src, dst, eattr, Ws1, Wn1, We1, b1)
    h = _mp(h, src, dst, eattr, Ws2, Wn2, We2, b2)
    return h @ Wo + bo


def _attn_body(hr_ref, hl_ref, rx_ref, a_ref, y_ref):
    hr = hr_ref[...]
    hl = hl_ref[...]
    dots = jax.lax.dot_general(hl, hr, (((1,), (1,)), ((), ())),
                               preferred_element_type=jnp.float32)
    m = jnp.max(dots, axis=1, keepdims=True)
    e = jnp.exp(dots - m)
    s = jnp.sum(e, axis=1, keepdims=True)
    a = e / s
    a_ref[...] = a
    y_ref[...] = jax.lax.dot_general(a, rx_ref[...], (((1,), (0,)), ((), ())),
                                     preferred_element_type=jnp.float32)


def _attention(h_r, h_l, rec_x):
    K = h_l.shape[0]
    N = h_r.shape[0]
    return pl.pallas_call(
        _attn_body,
        out_shape=(
            jax.ShapeDtypeStruct((K, N), jnp.float32),
            jax.ShapeDtypeStruct((K, rec_x.shape[1]), jnp.float32),
        ),
    )(h_r, h_l, rec_x)


def kernel(rec_attr, rec_x, rec_edge_index, rec_edge_attr, lig_attr,
           lig_edge_index, lig_edge_attr, labelidx, Wr_self1, Wr_nbr1, Wr_e1,
           br1, Wr_self2, Wr_nbr2, Wr_e2, br2, Wr_out, br_out, Wl_self1,
           Wl_nbr1, Wl_e1, bl1, Wl_self2, Wl_nbr2, Wl_e2, bl2, Wl_out,
           bl_out, phi_W, phi_b, Ascaler1, Ascaler2):
    hs_rec = _enc(rec_attr, rec_edge_index, rec_edge_attr,
                  (Wr_self1, Wr_nbr1, Wr_e1, br1, Wr_self2, Wr_nbr2, Wr_e2,
                   br2, Wr_out, br_out))
    hs_lig = _enc(lig_attr, lig_edge_index, lig_edge_attr,
                  (Wl_self1, Wl_nbr1, Wl_e1, bl1, Wl_self2, Wl_nbr2, Wl_e2,
                   bl2, Wl_out, bl_out))
    h_l = labelidx @ hs_lig
    h_l = jax.nn.relu(h_l @ phi_W + phi_b)
    A, Yrec = _attention(hs_rec, h_l, rec_x)
    return (Yrec[None], A)


# speedup vs baseline: 11.2761x; 11.2761x over previous
import functools

import jax
import jax.numpy as jnp
from jax import lax
from jax.experimental import pallas as pl
from jax.experimental.pallas import tpu as pltpu
from jax.experimental.pallas import tpu_sc as plsc

NC = 2
NS = 16
NW = NC * NS
CH = 128
D = 32
DH = D // NC


def _sc_body(nblocks, K, with_e, *refs):
    if with_e:
        (table, src3, dst2, ea2, zrows, zelem, out_rows, out_e,
         sbuf, dbuf, ebuf, rows, ebounce, acc, acce, gsem) = refs
    else:
        (table, src3, dst2, zrows, out_rows,
         sbuf, dbuf, rows, acc, gsem) = refs
    cid = lax.axis_index("c")
    sid = lax.axis_index("s")
    n_acc = acc.shape[0]
    rpt = n_acc // NS
    pltpu.sync_copy(zrows, acc.at[pl.ds(sid * rpt, rpt)])
    if with_e:
        pltpu.sync_copy(zelem, ebounce)
        pltpu.sync_copy(ebounce, acce.at[pl.ds(sid * rpt, rpt)])
    plsc.subcore_barrier()

    wbase = sid * (nblocks * K)

    def blk(b, carry):
        rowbase = wbase + b * K
        pltpu.sync_copy(src3.at[cid, pl.ds(rowbase, K)], sbuf)
        pltpu.sync_copy(dst2.at[pl.ds(rowbase, K)], dbuf)
        if with_e:
            pltpu.sync_copy(ea2.at[pl.ds(rowbase, K)], ebuf)
        for j in range(K):
            pltpu.async_copy(table.at[sbuf.at[j]], rows.at[j], gsem)
        for j in range(K):
            pltpu.make_async_copy(table.at[sbuf.at[j]], rows.at[j],
                                  gsem).wait()
            pltpu.sync_copy(rows.at[j], acc.at[dbuf.at[j]], add=True)
            if with_e:
                pltpu.sync_copy(ebuf.at[j], acce.at[dbuf.at[j]], add=True)
        return carry

    lax.fori_loop(0, nblocks, blk, 0)
    plsc.subcore_barrier()
    pltpu.sync_copy(acc.at[pl.ds(sid * rpt, rpt)],
                    out_rows.at[cid, pl.ds(sid * rpt, rpt)])
    if with_e:
        pltpu.sync_copy(acce.at[pl.ds(sid * rpt, rpt)], ebounce)
        pltpu.sync_copy(ebounce,
                        out_e.at[pl.ds(cid * n_acc + sid * rpt, rpt)])


def _make_seg_sum(n_acc, nblocks, K, with_e):
    mesh = plsc.VectorSubcoreMesh(core_axis_name="c", subcore_axis_name="s")
    rpt = n_acc // NS
    out_type = [jax.ShapeDtypeStruct((NC, n_acc, DH), jnp.float32)]
    scratch = [
        pltpu.VMEM((K, CH), jnp.int32),
        pltpu.VMEM((K, CH), jnp.int32),
    ]
    if with_e:
        out_type.append(jax.ShapeDtypeStruct((NC * n_acc,), jnp.float32))
        scratch.append(pltpu.VMEM((K, CH), jnp.float32))
    scratch.append(pltpu.VMEM((K, CH, DH), jnp.float32))
    if with_e:
        scratch.append(pltpu.VMEM((rpt,), jnp.float32))
    scratch.append(pltpu.VMEM_SHARED((n_acc, DH), jnp.float32))
    if with_e:
        scratch.append(pltpu.VMEM_SHARED((n_acc,), jnp.float32))
    scratch.append(pltpu.SemaphoreType.DMA)

    body = functools.partial(_sc_body, nblocks, K, with_e)
    return pl.kernel(body, out_type=tuple(out_type), mesh=mesh,
                     scratch_types=tuple(scratch),
                     compiler_params=pltpu.CompilerParams(
                         use_tc_tiling_on_sc=False))


def _seg_sum(table, src3, dst2, n_acc, nblocks, K, ea2=None):
    n = table.shape[0]
    tstk = table.reshape(n, NC, DH).transpose(1, 0, 2).reshape(NC * n, DH)
    f = _make_seg_sum(n_acc, nblocks, K, ea2 is not None)
    rpt = n_acc // NS
    zrows = jnp.zeros((rpt, DH), jnp.float32)
    if ea2 is not None:
        zelem = jnp.zeros((rpt,), jnp.float32)
        gp, se = f(tstk, src3, dst2, ea2, zrows, zelem)
        return gp, se.reshape(NC, n_acc)
    return f(tstk, src3, dst2, zrows)


def _tc_a_body(attr_ref, wn_ref, ws_ref, b_ref, m_ref, pre_ref):
    attr = attr_ref[...]
    m_ref[...] = jax.lax.dot_general(
        attr, wn_ref[...], (((1,), (0,)), ((), ())),
        preferred_element_type=jnp.float32)
    pre_ref[...] = jax.lax.dot_general(
        attr, ws_ref[...], (((1,), (0,)), ((), ())),
        preferred_element_type=jnp.float32) + b_ref[...]


def _tc_a(attr, Wn, Ws, b, blk):
    n, f = attr.shape
    grid = n // blk
    return pl.pallas_call(
        _tc_a_body,
        grid=(grid,),
        in_specs=[
            pl.BlockSpec((blk, f), lambda i: (i, 0)),
            pl.BlockSpec(Wn.shape, lambda i: (0, 0)),
            pl.BlockSpec(Ws.shape, lambda i: (0, 0)),
            pl.BlockSpec((1, D), lambda i: (0, 0)),
        ],
        out_specs=[
            pl.BlockSpec((blk, D), lambda i: (i, 0)),
            pl.BlockSpec((blk, D), lambda i: (i, 0)),
        ],
        out_shape=[
            jax.ShapeDtypeStruct((n, D), jnp.float32),
            jax.ShapeDtypeStruct((n, D), jnp.float32),
        ],
    )(attr, Wn, Ws, b.reshape(1, D))


def _tc_b_body(pre_ref, g_ref, s_ref, we1_ref, we2_ref, wn_ref, ws_ref,
               b2_ref, m2_ref, pre2_ref):
    g = jnp.concatenate([g_ref[0], g_ref[1]], axis=-1)
    s = s_ref[...]
    h1 = jax.nn.relu(pre_ref[...] + g + s * we1_ref[...])
    m2_ref[...] = jax.lax.dot_general(
        h1, wn_ref[...], (((1,), (0,)), ((), ())),
        preferred_element_type=jnp.float32)
    pre2_ref[...] = jax.lax.dot_general(
        h1, ws_ref[...], (((1,), (0,)), ((), ())),
        preferred_element_type=jnp.float32) + s * we2_ref[...] + b2_ref[...]


def _tc_b(pre1, g1p, sp, We1, We2, Wn2, Ws2, b2, blk):
    n = pre1.shape[0]
    grid = n // blk
    return pl.pallas_call(
        _tc_b_body,
        grid=(grid,),
        in_specs=[
            pl.BlockSpec((blk, D), lambda i: (i, 0)),
            pl.BlockSpec((NC, blk, DH), lambda i: (0, i, 0)),
            pl.BlockSpec((blk, 1), lambda i: (i, 0)),
            pl.BlockSpec((1, D), lambda i: (0, 0)),
            pl.BlockSpec((1, D), lambda i: (0, 0)),
            pl.BlockSpec((D, D), lambda i: (0, 0)),
            pl.BlockSpec((D, D), lambda i: (0, 0)),
            pl.BlockSpec((1, D), lambda i: (0, 0)),
        ],
        out_specs=[
            pl.BlockSpec((blk, D), lambda i: (i, 0)),
            pl.BlockSpec((blk, D), lambda i: (i, 0)),
        ],
        out_shape=[
            jax.ShapeDtypeStruct((n, D), jnp.float32),
            jax.ShapeDtypeStruct((n, D), jnp.float32),
        ],
    )(pre1, g1p, sp, We1.reshape(1, D), We2.reshape(1, D), Wn2,
      Ws2, b2.reshape(1, D))


def _tc_c_body(pre2_ref, g_ref, wo_ref, bo_ref, hr_ref):
    h2 = jax.nn.relu(pre2_ref[...]
                     + jnp.concatenate([g_ref[0], g_ref[1]], axis=-1))
    hr_ref[...] = jax.lax.dot_general(
        h2, wo_ref[...], (((1,), (0,)), ((), ())),
        preferred_element_type=jnp.float32) + bo_ref[...]


def _tc_c(pre2, g2p, Wo, bo, blk):
    n = pre2.shape[0]
    grid = n // blk
    return pl.pallas_call(
        _tc_c_body,
        grid=(grid,),
        in_specs=[
            pl.BlockSpec((blk, D), lambda i: (i, 0)),
            pl.BlockSpec((NC, blk, DH), lambda i: (0, i, 0)),
            pl.BlockSpec((D, D), lambda i: (0, 0)),
            pl.BlockSpec((1, D), lambda i: (0, 0)),
        ],
        out_specs=pl.BlockSpec((blk, D), lambda i: (i, 0)),
        out_shape=jax.ShapeDtypeStruct((n, D), jnp.float32),
    )(pre2, g2p, Wo, bo.reshape(1, D))


def _attn_body(hr_ref, hl_ref, rx_ref, a_ref, y_ref):
    hr = hr_ref[...]
    hl = hl_ref[...]
    dots = jax.lax.dot_general(hl, hr, (((1,), (1,)), ((), ())),
                               preferred_element_type=jnp.float32)
    m = jnp.max(dots, axis=1, keepdims=True)
    e = jnp.exp(dots - m)
    s = jnp.sum(e, axis=1, keepdims=True)
    a = e / s
    a_ref[...] = a
    y_ref[...] = jax.lax.dot_general(a, rx_ref[...], (((1,), (0,)), ((), ())),
                                     preferred_element_type=jnp.float32)


def _attention(h_r, h_l, rec_x):
    k = h_l.shape[0]
    n = h_r.shape[0]
    return pl.pallas_call(
        _attn_body,
        out_shape=(
            jax.ShapeDtypeStruct((k, n), jnp.float32),
            jax.ShapeDtypeStruct((k, rec_x.shape[1]), jnp.float32),
        ),
    )(h_r, h_l, rec_x)


def _pad_edges(src, dst, n_table, n_nodes, n_acc, e_pad, eattr=None):
    e = src.shape[0]
    npad = e_pad - e
    fill_src = (jnp.arange(npad, dtype=jnp.int32) % n_table)
    fill_dst = n_nodes + (jnp.arange(npad, dtype=jnp.int32)
                          % (n_acc - n_nodes))
    src2 = jnp.concatenate([src, fill_src]).reshape(-1, CH)
    src3 = jnp.stack([src2, src2 + n_table])
    dst2 = jnp.concatenate([dst, fill_dst]).reshape(-1, CH)
    if eattr is None:
        return src3, dst2
    ea2 = jnp.concatenate(
        [eattr.reshape(-1), jnp.zeros((npad,), jnp.float32)]).reshape(-1, CH)
    return src3, dst2, ea2


def _mp(h, src, dst, e, Wself, Wnbr, We, b):
    msg = jnp.take(h, src, axis=0) @ Wnbr + e @ We
    agg = jnp.zeros((h.shape[0], Wnbr.shape[1]), h.dtype).at[dst].add(msg)
    return jax.nn.relu(h @ Wself + agg + b)


def _enc(attr, ei, eattr, params):
    Ws1, Wn1, We1, b1, Ws2, Wn2, We2, b2, Wo, bo = params
    src, dst = ei[0], ei[1]
    h = _mp(attr, src, dst, eattr, Ws1, Wn1, We1, b1)
    h = _mp(h, src, dst, eattr, Ws2, Wn2, We2, b2)
    return h @ Wo + bo


REC_K = 8
REC_NBLOCKS = 98
REC_EPAD = NS * REC_NBLOCKS * REC_K * CH
REC_NACC = 50432


def kernel(rec_attr, rec_x, rec_edge_index, rec_edge_attr, lig_attr,
           lig_edge_index, lig_edge_attr, labelidx, Wr_self1, Wr_nbr1, Wr_e1,
           br1, Wr_self2, Wr_nbr2, Wr_e2, br2, Wr_out, br_out, Wl_self1,
           Wl_nbr1, Wl_e1, bl1, Wl_self2, Wl_nbr2, Wl_e2, bl2, Wl_out,
           bl_out, phi_W, phi_b, Ascaler1, Ascaler2):
    n_rec = rec_attr.shape[0]

    src3, dst2, ea2 = _pad_edges(rec_edge_index[0], rec_edge_index[1],
                                 n_rec, n_rec, REC_NACC, REC_EPAD,
                                 rec_edge_attr)
    m1, pre1 = _tc_a(rec_attr, Wr_nbr1, Wr_self1, br1, blk=5000)
    g1p, sep = _seg_sum(m1, src3, dst2, REC_NACC, REC_NBLOCKS, REC_K,
                        ea2=ea2)
    sp = sep[0][:, None]
    m2, pre2 = _tc_b(pre1, g1p, sp, Wr_e1[0], Wr_e2[0], Wr_nbr2, Wr_self2,
                     br2, blk=5000)
    (g2p,) = _seg_sum(m2, src3, dst2, REC_NACC, REC_NBLOCKS, REC_K)
    hs_rec = _tc_c(pre2, g2p, Wr_out, br_out, blk=5000)

    hs_lig = _enc(lig_attr, lig_edge_index, lig_edge_attr,
                  (Wl_self1, Wl_nbr1, Wl_e1, bl1, Wl_self2, Wl_nbr2, Wl_e2,
                   bl2, Wl_out, bl_out))

    h_l = labelidx @ hs_lig
    h_l = jax.nn.relu(h_l @ phi_W + phi_b)
    A, Yrec = _attention(hs_rec, h_l, rec_x)
    return (Yrec[None], A)


# speedup vs baseline: 11.4213x; 1.0129x over previous
import functools

import jax
import jax.numpy as jnp
from jax import lax
from jax.experimental import pallas as pl
from jax.experimental.pallas import tpu as pltpu
from jax.experimental.pallas import tpu_sc as plsc

NC = 2
NS = 16
NW = NC * NS
CH = 128
D = 32
DH = D // NC


def _sc_body(nblocks, K, with_e, *refs):
    if with_e:
        (table, src3, dst2, ea2, zrows, zelem, out_rows, out_e,
         sbuf, dbuf, ebuf, rows, ebounce, acc, acce, gsem) = refs
    else:
        (table, src3, dst2, zrows, out_rows,
         sbuf, dbuf, rows, acc, gsem) = refs
    cid = lax.axis_index("c")
    sid = lax.axis_index("s")
    n_acc = acc.shape[0]
    rpt = n_acc // NS
    pltpu.sync_copy(zrows, acc.at[pl.ds(sid * rpt, rpt)])
    if with_e:
        pltpu.sync_copy(zelem, ebounce)
        pltpu.sync_copy(ebounce, acce.at[pl.ds(sid * rpt, rpt)])
    plsc.subcore_barrier()

    wbase = sid * (nblocks * K)

    def blk(b, carry):
        rowbase = wbase + b * K
        pltpu.sync_copy(src3.at[cid, pl.ds(rowbase, K)], sbuf)
        pltpu.sync_copy(dst2.at[pl.ds(rowbase, K)], dbuf)
        if with_e:
            pltpu.sync_copy(ea2.at[pl.ds(rowbase, K)], ebuf)
        for j in range(K):
            pltpu.async_copy(table.at[sbuf.at[j]], rows.at[j], gsem)
        for j in range(K):
            pltpu.make_async_copy(table.at[sbuf.at[j]], rows.at[j],
                                  gsem).wait()
            pltpu.sync_copy(rows.at[j], acc.at[dbuf.at[j]], add=True)
            if with_e:
                pltpu.sync_copy(ebuf.at[j], acce.at[dbuf.at[j]], add=True)
        return carry

    lax.fori_loop(0, nblocks, blk, 0)
    plsc.subcore_barrier()
    pltpu.sync_copy(acc.at[pl.ds(sid * rpt, rpt)],
                    out_rows.at[cid, pl.ds(sid * rpt, rpt)])
    if with_e:
        pltpu.sync_copy(acce.at[pl.ds(sid * rpt, rpt)], ebounce)
        pltpu.sync_copy(ebounce,
                        out_e.at[pl.ds(cid * n_acc + sid * rpt, rpt)])


def _make_seg_sum(n_acc, nblocks, K, with_e):
    mesh = plsc.VectorSubcoreMesh(core_axis_name="c", subcore_axis_name="s")
    rpt = n_acc // NS
    out_type = [jax.ShapeDtypeStruct((NC, n_acc, DH), jnp.float32)]
    scratch = [
        pltpu.VMEM((K, CH), jnp.int32),
        pltpu.VMEM((K, CH), jnp.int32),
    ]
    if with_e:
        out_type.append(jax.ShapeDtypeStruct((NC * n_acc,), jnp.float32))
        scratch.append(pltpu.VMEM((K, CH), jnp.float32))
    scratch.append(pltpu.VMEM((K, CH, DH), jnp.float32))
    if with_e:
        scratch.append(pltpu.VMEM((rpt,), jnp.float32))
    scratch.append(pltpu.VMEM_SHARED((n_acc, DH), jnp.float32))
    if with_e:
        scratch.append(pltpu.VMEM_SHARED((n_acc,), jnp.float32))
    scratch.append(pltpu.SemaphoreType.DMA)

    body = functools.partial(_sc_body, nblocks, K, with_e)
    return pl.kernel(body, out_type=tuple(out_type), mesh=mesh,
                     scratch_types=tuple(scratch),
                     compiler_params=pltpu.CompilerParams(
                         use_tc_tiling_on_sc=False))


def _seg_sum(table, src3, dst2, n_acc, nblocks, K, ea2=None):
    n = table.shape[0]
    tstk = table.reshape(n, NC, DH).transpose(1, 0, 2).reshape(NC * n, DH)
    f = _make_seg_sum(n_acc, nblocks, K, ea2 is not None)
    rpt = n_acc // NS
    zrows = jnp.zeros((rpt, DH), jnp.float32)
    if ea2 is not None:
        zelem = jnp.zeros((rpt,), jnp.float32)
        gp, se = f(tstk, src3, dst2, ea2, zrows, zelem)
        return gp, se.reshape(NC, n_acc)
    return f(tstk, src3, dst2, zrows)


def _tc_a_body(attr_ref, wn_ref, ws_ref, b_ref, m_ref, pre_ref):
    attr = attr_ref[...]
    m_ref[...] = jax.lax.dot_general(
        attr, wn_ref[...], (((1,), (0,)), ((), ())),
        preferred_element_type=jnp.float32)
    pre_ref[...] = jax.lax.dot_general(
        attr, ws_ref[...], (((1,), (0,)), ((), ())),
        preferred_element_type=jnp.float32) + b_ref[...]


def _tc_a(attr, Wn, Ws, b, blk):
    n, f = attr.shape
    grid = n // blk
    return pl.pallas_call(
        _tc_a_body,
        grid=(grid,),
        in_specs=[
            pl.BlockSpec((blk, f), lambda i: (i, 0)),
            pl.BlockSpec(Wn.shape, lambda i: (0, 0)),
            pl.BlockSpec(Ws.shape, lambda i: (0, 0)),
            pl.BlockSpec((1, D), lambda i: (0, 0)),
        ],
        out_specs=[
            pl.BlockSpec((blk, D), lambda i: (i, 0)),
            pl.BlockSpec((blk, D), lambda i: (i, 0)),
        ],
        out_shape=[
            jax.ShapeDtypeStruct((n, D), jnp.float32),
            jax.ShapeDtypeStruct((n, D), jnp.float32),
        ],
    )(attr, Wn, Ws, b.reshape(1, D))


def _tc_b_body(pre_ref, g_ref, s_ref, we1_ref, we2_ref, wn_ref, ws_ref,
               b2_ref, m2_ref, pre2_ref):
    g = jnp.concatenate([g_ref[0], g_ref[1]], axis=-1)
    s = s_ref[...]
    h1 = jax.nn.relu(pre_ref[...] + g + s * we1_ref[...])
    m2_ref[...] = jax.lax.dot_general(
        h1, wn_ref[...], (((1,), (0,)), ((), ())),
        preferred_element_type=jnp.float32)
    pre2_ref[...] = jax.lax.dot_general(
        h1, ws_ref[...], (((1,), (0,)), ((), ())),
        preferred_element_type=jnp.float32) + s * we2_ref[...] + b2_ref[...]


def _tc_b(pre1, g1p, sp, We1, We2, Wn2, Ws2, b2, blk):
    n = pre1.shape[0]
    grid = n // blk
    return pl.pallas_call(
        _tc_b_body,
        grid=(grid,),
        in_specs=[
            pl.BlockSpec((blk, D), lambda i: (i, 0)),
            pl.BlockSpec((NC, blk, DH), lambda i: (0, i, 0)),
            pl.BlockSpec((blk, 1), lambda i: (i, 0)),
            pl.BlockSpec((1, D), lambda i: (0, 0)),
            pl.BlockSpec((1, D), lambda i: (0, 0)),
            pl.BlockSpec((D, D), lambda i: (0, 0)),
            pl.BlockSpec((D, D), lambda i: (0, 0)),
            pl.BlockSpec((1, D), lambda i: (0, 0)),
        ],
        out_specs=[
            pl.BlockSpec((blk, D), lambda i: (i, 0)),
            pl.BlockSpec((blk, D), lambda i: (i, 0)),
        ],
        out_shape=[
            jax.ShapeDtypeStruct((n, D), jnp.float32),
            jax.ShapeDtypeStruct((n, D), jnp.float32),
        ],
    )(pre1, g1p, sp, We1.reshape(1, D), We2.reshape(1, D), Wn2,
      Ws2, b2.reshape(1, D))


def _tc_c_body(pre2_ref, g_ref, wo_ref, bo_ref, hr_ref):
    h2 = jax.nn.relu(pre2_ref[...]
                     + jnp.concatenate([g_ref[0], g_ref[1]], axis=-1))
    hr_ref[...] = jax.lax.dot_general(
        h2, wo_ref[...], (((1,), (0,)), ((), ())),
        preferred_element_type=jnp.float32) + bo_ref[...]


def _tc_c(pre2, g2p, Wo, bo, blk):
    n = pre2.shape[0]
    grid = n // blk
    return pl.pallas_call(
        _tc_c_body,
        grid=(grid,),
        in_specs=[
            pl.BlockSpec((blk, D), lambda i: (i, 0)),
            pl.BlockSpec((NC, blk, DH), lambda i: (0, i, 0)),
            pl.BlockSpec((D, D), lambda i: (0, 0)),
            pl.BlockSpec((1, D), lambda i: (0, 0)),
        ],
        out_specs=pl.BlockSpec((blk, D), lambda i: (i, 0)),
        out_shape=jax.ShapeDtypeStruct((n, D), jnp.float32),
    )(pre2, g2p, Wo, bo.reshape(1, D))


def _mm(a, b):
    return jax.lax.dot_general(a, b, (((1,), (0,)), ((), ())),
                               preferred_element_type=jnp.float32)


def _lig_a_body(attr_ref, eattr_ref, wn_ref, ws_ref, b_ref, we1_ref,
                we2_ref, m_ref, pre_ref, e1_ref, e2_ref):
    attr = attr_ref[...]
    ea = eattr_ref[...]
    m_ref[...] = _mm(attr, wn_ref[...])
    pre_ref[...] = _mm(attr, ws_ref[...]) + b_ref[...]
    e1_ref[...] = _mm(ea, we1_ref[...])
    e2_ref[...] = _mm(ea, we2_ref[...])


def _lig_a(attr, eattr, Wn, Ws, b, We1, We2):
    n = attr.shape[0]
    ne = eattr.shape[0]
    return pl.pallas_call(
        _lig_a_body,
        out_shape=[
            jax.ShapeDtypeStruct((n, D), jnp.float32),
            jax.ShapeDtypeStruct((n, D), jnp.float32),
            jax.ShapeDtypeStruct((ne, D), jnp.float32),
            jax.ShapeDtypeStruct((ne, D), jnp.float32),
        ],
    )(attr, eattr, Wn, Ws, b.reshape(1, D), We1, We2)


def _lig_b_body(pre_ref, g_ref, wn_ref, ws_ref, b2_ref, m2_ref, pre2_ref):
    n = pre_ref.shape[0]
    g = jnp.concatenate([g_ref[0, :n], g_ref[1, :n]], axis=-1)
    h1 = jax.nn.relu(pre_ref[...] + g)
    m2_ref[...] = _mm(h1, wn_ref[...])
    pre2_ref[...] = _mm(h1, ws_ref[...]) + b2_ref[...]


def _lig_b(pre1, g1p, Wn2, Ws2, b2):
    n = pre1.shape[0]
    return pl.pallas_call(
        _lig_b_body,
        out_shape=[
            jax.ShapeDtypeStruct((n, D), jnp.float32),
            jax.ShapeDtypeStruct((n, D), jnp.float32),
        ],
    )(pre1, g1p, Wn2, Ws2, b2.reshape(1, D))


def _final_body(pre2l_ref, gl_ref, wo_ref, bo_ref, lab_ref, phiw_ref,
                phib_ref, hr_ref, rx_ref, a_ref, y_ref):
    n = pre2l_ref.shape[0]
    gl = jnp.concatenate([gl_ref[0, :n], gl_ref[1, :n]], axis=-1)
    h2 = jax.nn.relu(pre2l_ref[...] + gl)
    hs_lig = _mm(h2, wo_ref[...]) + bo_ref[...]
    hl8 = _mm(lab_ref[...], hs_lig)
    hl = jax.nn.relu(_mm(hl8, phiw_ref[...]) + phib_ref[...])
    hr = hr_ref[...]
    dots = jax.lax.dot_general(hl, hr, (((1,), (1,)), ((), ())),
                               preferred_element_type=jnp.float32)
    m = jnp.max(dots, axis=1, keepdims=True)
    e = jnp.exp(dots - m)
    s = jnp.sum(e, axis=1, keepdims=True)
    a = e / s
    a_ref[...] = a
    y_ref[...] = jax.lax.dot_general(a, rx_ref[...], (((1,), (0,)), ((), ())),
                                     preferred_element_type=jnp.float32)


def _final(pre2l, g2pl, Wo, bo, labelidx, phi_W, phi_b, h_r, rec_x):
    k = labelidx.shape[0]
    n = h_r.shape[0]
    return pl.pallas_call(
        _final_body,
        out_shape=(
            jax.ShapeDtypeStruct((k, n), jnp.float32),
            jax.ShapeDtypeStruct((k, rec_x.shape[1]), jnp.float32),
        ),
    )(pre2l, g2pl, Wo, bo.reshape(1, D), labelidx, phi_W,
      phi_b.reshape(1, D), h_r, rec_x)


def _pad_edges(src, dst, n_table, n_nodes, n_acc, e_pad, eattr=None):
    e = src.shape[0]
    npad = e_pad - e
    fill_src = (jnp.arange(npad, dtype=jnp.int32) % n_table)
    fill_dst = n_nodes + (jnp.arange(npad, dtype=jnp.int32)
                          % (n_acc - n_nodes))
    src2 = jnp.concatenate([src, fill_src]).reshape(-1, CH)
    src3 = jnp.stack([src2, src2 + n_table])
    dst2 = jnp.concatenate([dst, fill_dst]).reshape(-1, CH)
    if eattr is None:
        return src3, dst2
    ea2 = jnp.concatenate(
        [eattr.reshape(-1), jnp.zeros((npad,), jnp.float32)]).reshape(-1, CH)
    return src3, dst2, ea2


REC_K = 8
REC_NBLOCKS = 98
REC_EPAD = NS * REC_NBLOCKS * REC_K * CH
REC_NACC = 50432

LIG_K = 8
LIG_NBLOCKS = 2
LIG_EPAD = NS * LIG_NBLOCKS * LIG_K * CH
LIG_NACC = 1024


def kernel(rec_attr, rec_x, rec_edge_index, rec_edge_attr, lig_attr,
           lig_edge_index, lig_edge_attr, labelidx, Wr_self1, Wr_nbr1, Wr_e1,
           br1, Wr_self2, Wr_nbr2, Wr_e2, br2, Wr_out, br_out, Wl_self1,
           Wl_nbr1, Wl_e1, bl1, Wl_self2, Wl_nbr2, Wl_e2, bl2, Wl_out,
           bl_out, phi_W, phi_b, Ascaler1, Ascaler2):
    n_rec = rec_attr.shape[0]

    src3, dst2, ea2 = _pad_edges(rec_edge_index[0], rec_edge_index[1],
                                 n_rec, n_rec, REC_NACC, REC_EPAD,
                                 rec_edge_attr)
    m1, pre1 = _tc_a(rec_attr, Wr_nbr1, Wr_self1, br1, blk=5000)
    g1p, sep = _seg_sum(m1, src3, dst2, REC_NACC, REC_NBLOCKS, REC_K,
                        ea2=ea2)
    sp = sep[0][:, None]
    m2, pre2 = _tc_b(pre1, g1p, sp, Wr_e1[0], Wr_e2[0], Wr_nbr2, Wr_self2,
                     br2, blk=5000)
    (g2p,) = _seg_sum(m2, src3, dst2, REC_NACC, REC_NBLOCKS, REC_K)
    hs_rec = _tc_c(pre2, g2p, Wr_out, br_out, blk=5000)

    n_lig = lig_attr.shape[0]
    e_lig = lig_edge_attr.shape[0]
    virt = n_lig + jnp.arange(e_lig, dtype=jnp.int32)
    asrc = jnp.concatenate([lig_edge_index[0], virt])
    adst = jnp.concatenate([lig_edge_index[1], lig_edge_index[1]])
    lsrc3, ldst2 = _pad_edges(asrc, adst, n_lig + e_lig, n_lig, LIG_NACC,
                              LIG_EPAD)
    lm1, lpre1, le1, le2 = _lig_a(lig_attr, lig_edge_attr, Wl_nbr1, Wl_self1,
                                  bl1, Wl_e1, Wl_e2)
    t1 = jnp.concatenate([lm1, le1], axis=0)
    (lg1p,) = _seg_sum(t1, lsrc3, ldst2, LIG_NACC, LIG_NBLOCKS, LIG_K)
    lm2, lpre2 = _lig_b(lpre1, lg1p, Wl_nbr2, Wl_self2, bl2)
    t2 = jnp.concatenate([lm2, le2], axis=0)
    (lg2p,) = _seg_sum(t2, lsrc3, ldst2, LIG_NACC, LIG_NBLOCKS, LIG_K)

    A, Yrec = _final(lpre2, lg2p, Wl_out, bl_out, labelidx, phi_W, phi_b,
                     hs_rec, rec_x)
    return (Yrec[None], A)


# speedup vs baseline: 13.1148x; 1.1483x over previous
import functools

import jax
import jax.numpy as jnp
from jax import lax
from jax.experimental import pallas as pl
from jax.experimental.pallas import tpu as pltpu
from jax.experimental.pallas import tpu_sc as plsc

NC = 2
NS = 16
NW = NC * NS
CH = 128
D = 32
DH = D // NC


def _sc_body(nblocks, K, with_e, *refs):
    if with_e:
        (table, src3, dst2, ea2, zrows, zelem, out_rows, out_e,
         sbuf, dbuf, ebuf, rows, ebounce, acc, acce,
         gsem, ssem, esem) = refs
    else:
        (table, src3, dst2, zrows, out_rows,
         sbuf, dbuf, rows, acc, gsem, ssem) = refs
    cid = lax.axis_index("c")
    sid = lax.axis_index("s")
    n_acc = acc.shape[0]
    rpt = n_acc // NS
    pltpu.sync_copy(zrows, acc.at[pl.ds(sid * rpt, rpt)])
    if with_e:
        pltpu.sync_copy(zelem, ebounce)
        pltpu.sync_copy(ebounce, acce.at[pl.ds(sid * rpt, rpt)])
    plsc.subcore_barrier()

    wbase = sid * (nblocks * K)

    def load_and_fire(bn, q):
        rowbase = wbase + bn * K
        pltpu.sync_copy(src3.at[cid, pl.ds(rowbase, K)], sbuf.at[q])
        pltpu.sync_copy(dst2.at[pl.ds(rowbase, K)], dbuf.at[q])
        if with_e:
            pltpu.sync_copy(ea2.at[pl.ds(rowbase, K)], ebuf.at[q])
        for j in range(K):
            pltpu.async_copy(table.at[sbuf.at[q, j]], rows.at[q, j], gsem)

    def drain(q):
        for j in range(K):
            pltpu.make_async_copy(rows.at[q, j], acc.at[dbuf.at[q, j]],
                                  ssem).wait()
            if with_e:
                pltpu.make_async_copy(ebuf.at[q, j], acce.at[dbuf.at[q, j]],
                                      esem).wait()

    def process(b, p):
        q = 1 - p
        for j in range(K):
            pltpu.make_async_copy(table.at[sbuf.at[p, j]], rows.at[p, j],
                                  gsem).wait()
            pltpu.async_copy(rows.at[p, j], acc.at[dbuf.at[p, j]], ssem,
                             add=True)
            if with_e:
                pltpu.async_copy(ebuf.at[p, j], acce.at[dbuf.at[p, j]],
                                 esem, add=True)
        @pl.when(b > 0)
        def _():
            drain(q)

        @pl.when(b + 1 < nblocks)
        def _():
            load_and_fire(b + 1, q)

    load_and_fire(0, 0)

    def blk2(i, carry):
        process(2 * i, 0)
        process(2 * i + 1, 1)
        return carry

    lax.fori_loop(0, nblocks // 2, blk2, 0)
    drain((nblocks - 1) & 1)
    plsc.subcore_barrier()
    pltpu.sync_copy(acc.at[pl.ds(sid * rpt, rpt)],
                    out_rows.at[cid, pl.ds(sid * rpt, rpt)])
    if with_e:
        pltpu.sync_copy(acce.at[pl.ds(sid * rpt, rpt)], ebounce)
        pltpu.sync_copy(ebounce,
                        out_e.at[pl.ds(cid * n_acc + sid * rpt, rpt)])


def _make_seg_sum(n_acc, nblocks, K, with_e):
    mesh = plsc.VectorSubcoreMesh(core_axis_name="c", subcore_axis_name="s")
    rpt = n_acc // NS
    out_type = [jax.ShapeDtypeStruct((NC, n_acc, DH), jnp.float32)]
    scratch = [
        pltpu.VMEM((2, K, CH), jnp.int32),
        pltpu.VMEM((2, K, CH), jnp.int32),
    ]
    if with_e:
        out_type.append(jax.ShapeDtypeStruct((NC * n_acc,), jnp.float32))
        scratch.append(pltpu.VMEM((2, K, CH), jnp.float32))
    scratch.append(pltpu.VMEM((2, K, CH, DH), jnp.float32))
    if with_e:
        scratch.append(pltpu.VMEM((rpt,), jnp.float32))
    scratch.append(pltpu.VMEM_SHARED((n_acc, DH), jnp.float32))
    if with_e:
        scratch.append(pltpu.VMEM_SHARED((n_acc,), jnp.float32))
    scratch.append(pltpu.SemaphoreType.DMA)
    scratch.append(pltpu.SemaphoreType.DMA)
    if with_e:
        scratch.append(pltpu.SemaphoreType.DMA)

    body = functools.partial(_sc_body, nblocks, K, with_e)
    return pl.kernel(body, out_type=tuple(out_type), mesh=mesh,
                     scratch_types=tuple(scratch),
                     compiler_params=pltpu.CompilerParams(
                         use_tc_tiling_on_sc=False))


def _seg_sum(table, src3, dst2, n_acc, nblocks, K, ea2=None):
    n = table.shape[0]
    tstk = table.reshape(n, NC, DH).transpose(1, 0, 2).reshape(NC * n, DH)
    f = _make_seg_sum(n_acc, nblocks, K, ea2 is not None)
    rpt = n_acc // NS
    zrows = jnp.zeros((rpt, DH), jnp.float32)
    if ea2 is not None:
        zelem = jnp.zeros((rpt,), jnp.float32)
        gp, se = f(tstk, src3, dst2, ea2, zrows, zelem)
        return gp, se.reshape(NC, n_acc)
    return f(tstk, src3, dst2, zrows)


def _tc_a_body(attr_ref, wn_ref, ws_ref, b_ref, m_ref, pre_ref):
    attr = attr_ref[...]
    m_ref[...] = jax.lax.dot_general(
        attr, wn_ref[...], (((1,), (0,)), ((), ())),
        preferred_element_type=jnp.float32)
    pre_ref[...] = jax.lax.dot_general(
        attr, ws_ref[...], (((1,), (0,)), ((), ())),
        preferred_element_type=jnp.float32) + b_ref[...]


def _tc_a(attr, Wn, Ws, b, blk):
    n, f = attr.shape
    grid = n // blk
    return pl.pallas_call(
        _tc_a_body,
        grid=(grid,),
        in_specs=[
            pl.BlockSpec((blk, f), lambda i: (i, 0)),
            pl.BlockSpec(Wn.shape, lambda i: (0, 0)),
            pl.BlockSpec(Ws.shape, lambda i: (0, 0)),
            pl.BlockSpec((1, D), lambda i: (0, 0)),
        ],
        out_specs=[
            pl.BlockSpec((blk, D), lambda i: (i, 0)),
            pl.BlockSpec((blk, D), lambda i: (i, 0)),
        ],
        out_shape=[
            jax.ShapeDtypeStruct((n, D), jnp.float32),
            jax.ShapeDtypeStruct((n, D), jnp.float32),
        ],
    )(attr, Wn, Ws, b.reshape(1, D))


def _tc_b_body(pre_ref, g_ref, s_ref, we1_ref, we2_ref, wn_ref, ws_ref,
               b2_ref, m2_ref, pre2_ref):
    g = jnp.concatenate([g_ref[0], g_ref[1]], axis=-1)
    s = s_ref[...]
    h1 = jax.nn.relu(pre_ref[...] + g + s * we1_ref[...])
    m2_ref[...] = jax.lax.dot_general(
        h1, wn_ref[...], (((1,), (0,)), ((), ())),
        preferred_element_type=jnp.float32)
    pre2_ref[...] = jax.lax.dot_general(
        h1, ws_ref[...], (((1,), (0,)), ((), ())),
        preferred_element_type=jnp.float32) + s * we2_ref[...] + b2_ref[...]


def _tc_b(pre1, g1p, sp, We1, We2, Wn2, Ws2, b2, blk):
    n = pre1.shape[0]
    grid = n // blk
    return pl.pallas_call(
        _tc_b_body,
        grid=(grid,),
        in_specs=[
            pl.BlockSpec((blk, D), lambda i: (i, 0)),
            pl.BlockSpec((NC, blk, DH), lambda i: (0, i, 0)),
            pl.BlockSpec((blk, 1), lambda i: (i, 0)),
            pl.BlockSpec((1, D), lambda i: (0, 0)),
            pl.BlockSpec((1, D), lambda i: (0, 0)),
            pl.BlockSpec((D, D), lambda i: (0, 0)),
            pl.BlockSpec((D, D), lambda i: (0, 0)),
            pl.BlockSpec((1, D), lambda i: (0, 0)),
        ],
        out_specs=[
            pl.BlockSpec((blk, D), lambda i: (i, 0)),
            pl.BlockSpec((blk, D), lambda i: (i, 0)),
        ],
        out_shape=[
            jax.ShapeDtypeStruct((n, D), jnp.float32),
            jax.ShapeDtypeStruct((n, D), jnp.float32),
        ],
    )(pre1, g1p, sp, We1.reshape(1, D), We2.reshape(1, D), Wn2,
      Ws2, b2.reshape(1, D))


def _tc_c_body(pre2_ref, g_ref, wo_ref, bo_ref, hr_ref):
    h2 = jax.nn.relu(pre2_ref[...]
                     + jnp.concatenate([g_ref[0], g_ref[1]], axis=-1))
    hr_ref[...] = jax.lax.dot_general(
        h2, wo_ref[...], (((1,), (0,)), ((), ())),
        preferred_element_type=jnp.float32) + bo_ref[...]


def _tc_c(pre2, g2p, Wo, bo, blk):
    n = pre2.shape[0]
    grid = n // blk
    return pl.pallas_call(
        _tc_c_body,
        grid=(grid,),
        in_specs=[
            pl.BlockSpec((blk, D), lambda i: (i, 0)),
            pl.BlockSpec((NC, blk, DH), lambda i: (0, i, 0)),
            pl.BlockSpec((D, D), lambda i: (0, 0)),
            pl.BlockSpec((1, D), lambda i: (0, 0)),
        ],
        out_specs=pl.BlockSpec((blk, D), lambda i: (i, 0)),
        out_shape=jax.ShapeDtypeStruct((n, D), jnp.float32),
    )(pre2, g2p, Wo, bo.reshape(1, D))


def _mm(a, b):
    return jax.lax.dot_general(a, b, (((1,), (0,)), ((), ())),
                               preferred_element_type=jnp.float32)


def _lig_a_body(attr_ref, eattr_ref, wn_ref, ws_ref, b_ref, we1_ref,
                we2_ref, m_ref, pre_ref, e1_ref, e2_ref):
    attr = attr_ref[...]
    ea = eattr_ref[...]
    m_ref[...] = _mm(attr, wn_ref[...])
    pre_ref[...] = _mm(attr, ws_ref[...]) + b_ref[...]
    e1_ref[...] = _mm(ea, we1_ref[...])
    e2_ref[...] = _mm(ea, we2_ref[...])


def _lig_a(attr, eattr, Wn, Ws, b, We1, We2):
    n = attr.shape[0]
    ne = eattr.shape[0]
    return pl.pallas_call(
        _lig_a_body,
        out_shape=[
            jax.ShapeDtypeStruct((n, D), jnp.float32),
            jax.ShapeDtypeStruct((n, D), jnp.float32),
            jax.ShapeDtypeStruct((ne, D), jnp.float32),
            jax.ShapeDtypeStruct((ne, D), jnp.float32),
        ],
    )(attr, eattr, Wn, Ws, b.reshape(1, D), We1, We2)


def _lig_b_body(pre_ref, g_ref, wn_ref, ws_ref, b2_ref, m2_ref, pre2_ref):
    n = pre_ref.shape[0]
    g = jnp.concatenate([g_ref[0, :n], g_ref[1, :n]], axis=-1)
    h1 = jax.nn.relu(pre_ref[...] + g)
    m2_ref[...] = _mm(h1, wn_ref[...])
    pre2_ref[...] = _mm(h1, ws_ref[...]) + b2_ref[...]


def _lig_b(pre1, g1p, Wn2, Ws2, b2):
    n = pre1.shape[0]
    return pl.pallas_call(
        _lig_b_body,
        out_shape=[
            jax.ShapeDtypeStruct((n, D), jnp.float32),
            jax.ShapeDtypeStruct((n, D), jnp.float32),
        ],
    )(pre1, g1p, Wn2, Ws2, b2.reshape(1, D))


def _final_body(pre2l_ref, gl_ref, wo_ref, bo_ref, lab_ref, phiw_ref,
                phib_ref, hr_ref, rx_ref, a_ref, y_ref):
    n = pre2l_ref.shape[0]
    gl = jnp.concatenate([gl_ref[0, :n], gl_ref[1, :n]], axis=-1)
    h2 = jax.nn.relu(pre2l_ref[...] + gl)
    hs_lig = _mm(h2, wo_ref[...]) + bo_ref[...]
    hl8 = _mm(lab_ref[...], hs_lig)
    hl = jax.nn.relu(_mm(hl8, phiw_ref[...]) + phib_ref[...])
    hr = hr_ref[...]
    dots = jax.lax.dot_general(hl, hr, (((1,), (1,)), ((), ())),
                               preferred_element_type=jnp.float32)
    m = jnp.max(dots, axis=1, keepdims=True)
    e = jnp.exp(dots - m)
    s = jnp.sum(e, axis=1, keepdims=True)
    a = e / s
    a_ref[...] = a
    y_ref[...] = jax.lax.dot_general(a, rx_ref[...], (((1,), (0,)), ((), ())),
                                     preferred_element_type=jnp.float32)


def _final(pre2l, g2pl, Wo, bo, labelidx, phi_W, phi_b, h_r, rec_x):
    k = labelidx.shape[0]
    n = h_r.shape[0]
    return pl.pallas_call(
        _final_body,
        out_shape=(
            jax.ShapeDtypeStruct((k, n), jnp.float32),
            jax.ShapeDtypeStruct((k, rec_x.shape[1]), jnp.float32),
        ),
    )(pre2l, g2pl, Wo, bo.reshape(1, D), labelidx, phi_W,
      phi_b.reshape(1, D), h_r, rec_x)


def _pad_edges(src, dst, n_table, n_nodes, n_acc, e_pad, eattr=None):
    e = src.shape[0]
    npad = e_pad - e
    fill_src = (jnp.arange(npad, dtype=jnp.int32) % n_table)
    fill_dst = n_nodes + (jnp.arange(npad, dtype=jnp.int32)
                          % (n_acc - n_nodes))
    src2 = jnp.concatenate([src, fill_src]).reshape(-1, CH)
    src3 = jnp.stack([src2, src2 + n_table])
    dst2 = jnp.concatenate([dst, fill_dst]).reshape(-1, CH)
    if eattr is None:
        return src3, dst2
    ea2 = jnp.concatenate(
        [eattr.reshape(-1), jnp.zeros((npad,), jnp.float32)]).reshape(-1, CH)
    return src3, dst2, ea2


REC_K = 8
REC_NBLOCKS = 98
REC_EPAD = NS * REC_NBLOCKS * REC_K * CH
REC_NACC = 50432

LIG_K = 8
LIG_NBLOCKS = 2
LIG_EPAD = NS * LIG_NBLOCKS * LIG_K * CH
LIG_NACC = 1024


def kernel(rec_attr, rec_x, rec_edge_index, rec_edge_attr, lig_attr,
           lig_edge_index, lig_edge_attr, labelidx, Wr_self1, Wr_nbr1, Wr_e1,
           br1, Wr_self2, Wr_nbr2, Wr_e2, br2, Wr_out, br_out, Wl_self1,
           Wl_nbr1, Wl_e1, bl1, Wl_self2, Wl_nbr2, Wl_e2, bl2, Wl_out,
           bl_out, phi_W, phi_b, Ascaler1, Ascaler2):
    n_rec = rec_attr.shape[0]

    src3, dst2, ea2 = _pad_edges(rec_edge_index[0], rec_edge_index[1],
                                 n_rec, n_rec, REC_NACC, REC_EPAD,
                                 rec_edge_attr)
    m1, pre1 = _tc_a(rec_attr, Wr_nbr1, Wr_self1, br1, blk=5000)
    g1p, sep = _seg_sum(m1, src3, dst2, REC_NACC, REC_NBLOCKS, REC_K,
                        ea2=ea2)
    sp = sep[0][:, None]
    m2, pre2 = _tc_b(pre1, g1p, sp, Wr_e1[0], Wr_e2[0], Wr_nbr2, Wr_self2,
                     br2, blk=5000)
    (g2p,) = _seg_sum(m2, src3, dst2, REC_NACC, REC_NBLOCKS, REC_K)
    hs_rec = _tc_c(pre2, g2p, Wr_out, br_out, blk=5000)

    n_lig = lig_attr.shape[0]
    e_lig = lig_edge_attr.shape[0]
    virt = n_lig + jnp.arange(e_lig, dtype=jnp.int32)
    asrc = jnp.concatenate([lig_edge_index[0], virt])
    adst = jnp.concatenate([lig_edge_index[1], lig_edge_index[1]])
    lsrc3, ldst2 = _pad_edges(asrc, adst, n_lig + e_lig, n_lig, LIG_NACC,
                              LIG_EPAD)
    lm1, lpre1, le1, le2 = _lig_a(lig_attr, lig_edge_attr, Wl_nbr1, Wl_self1,
                                  bl1, Wl_e1, Wl_e2)
    t1 = jnp.concatenate([lm1, le1], axis=0)
    (lg1p,) = _seg_sum(t1, lsrc3, ldst2, LIG_NACC, LIG_NBLOCKS, LIG_K)
    lm2, lpre2 = _lig_b(lpre1, lg1p, Wl_nbr2, Wl_self2, bl2)
    t2 = jnp.concatenate([lm2, le2], axis=0)
    (lg2p,) = _seg_sum(t2, lsrc3, ldst2, LIG_NACC, LIG_NBLOCKS, LIG_K)

    A, Yrec = _final(lpre2, lg2p, Wl_out, bl_out, labelidx, phi_W, phi_b,
                     hs_rec, rec_x)
    return (Yrec[None], A)


# speedup vs baseline: 14.8547x; 1.1327x over previous
import functools

import jax
import jax.numpy as jnp
from jax import lax
from jax.experimental import pallas as pl
from jax.experimental.pallas import tpu as pltpu
from jax.experimental.pallas import tpu_sc as plsc

NC = 2
NS = 16
NW = NC * NS
CH = 128
D = 32
DH = D // NC


def _sc_body(n_table, nblocks, K, with_e, *refs):
    if with_e:
        (table, ei3, ea3, zrows, zelem, out_rows, out_e,
         sbuf, dbuf, ebuf, rows, ebounce, acc, acce,
         gsem, ssem, esem) = refs
    else:
        (table, ei3, zrows, out_rows,
         sbuf, dbuf, rows, acc, gsem, ssem) = refs
    cid = lax.axis_index("c")
    sid = lax.axis_index("s")
    n_acc = acc.shape[0]
    rpt = n_acc // NS
    pltpu.sync_copy(zrows, acc.at[pl.ds(sid * rpt, rpt)])
    if with_e:
        pltpu.sync_copy(zelem, ebounce)
        pltpu.sync_copy(ebounce, acce.at[pl.ds(sid * rpt, rpt)])
    plsc.subcore_barrier()

    wbase = sid * (nblocks * K)

    off = jnp.full((16,), cid * n_table, jnp.int32)

    def load_and_fire(bn, q):
        rowbase = wbase + bn * K
        pltpu.sync_copy(ei3.at[0, pl.ds(rowbase, K)], sbuf.at[q])
        pltpu.sync_copy(ei3.at[1, pl.ds(rowbase, K)], dbuf.at[q])
        if with_e:
            pltpu.sync_copy(ea3.at[pl.ds(rowbase, K)], ebuf.at[q])
        for j in range(K):
            for i in range(CH // 16):
                sl = pl.ds(i * 16, 16)
                sbuf[q, j, sl] = sbuf[q, j, sl] + off
        for j in range(K):
            pltpu.async_copy(table.at[sbuf.at[q, j]], rows.at[q, j], gsem)

    def drain(q):
        for j in range(K):
            pltpu.make_async_copy(rows.at[q, j], acc.at[dbuf.at[q, j]],
                                  ssem).wait()
            if with_e:
                pltpu.make_async_copy(ebuf.at[q, j], acce.at[dbuf.at[q, j]],
                                      esem).wait()

    def process(b, p):
        q = 1 - p
        for j in range(K):
            pltpu.make_async_copy(table.at[sbuf.at[p, j]], rows.at[p, j],
                                  gsem).wait()
            pltpu.async_copy(rows.at[p, j], acc.at[dbuf.at[p, j]], ssem,
                             add=True)
            if with_e:
                pltpu.async_copy(ebuf.at[p, j], acce.at[dbuf.at[p, j]],
                                 esem, add=True)
        @pl.when(b > 0)
        def _():
            drain(q)

        @pl.when(b + 1 < nblocks)
        def _():
            load_and_fire(b + 1, q)

    load_and_fire(0, 0)

    def blk2(i, carry):
        process(2 * i, 0)
        process(2 * i + 1, 1)
        return carry

    lax.fori_loop(0, nblocks // 2, blk2, 0)
    drain((nblocks - 1) & 1)
    plsc.subcore_barrier()
    pltpu.sync_copy(acc.at[pl.ds(sid * rpt, rpt)],
                    out_rows.at[cid, pl.ds(sid * rpt, rpt)])
    if with_e:
        pltpu.sync_copy(acce.at[pl.ds(sid * rpt, rpt)], ebounce)
        pltpu.sync_copy(ebounce,
                        out_e.at[pl.ds(cid * n_acc + sid * rpt, rpt)])


def _make_seg_sum(n_table, n_acc, nblocks, K, with_e):
    mesh = plsc.VectorSubcoreMesh(core_axis_name="c", subcore_axis_name="s")
    rpt = n_acc // NS
    out_type = [jax.ShapeDtypeStruct((NC, n_acc, DH), jnp.float32)]
    scratch = [
        pltpu.VMEM((2, K, CH), jnp.int32),
        pltpu.VMEM((2, K, CH), jnp.int32),
    ]
    if with_e:
        out_type.append(jax.ShapeDtypeStruct((NC * n_acc,), jnp.float32))
        scratch.append(pltpu.VMEM((2, K, CH), jnp.float32))
    scratch.append(pltpu.VMEM((2, K, CH, DH), jnp.float32))
    if with_e:
        scratch.append(pltpu.VMEM((rpt,), jnp.float32))
    scratch.append(pltpu.VMEM_SHARED((n_acc, DH), jnp.float32))
    if with_e:
        scratch.append(pltpu.VMEM_SHARED((n_acc,), jnp.float32))
    scratch.append(pltpu.SemaphoreType.DMA)
    scratch.append(pltpu.SemaphoreType.DMA)
    if with_e:
        scratch.append(pltpu.SemaphoreType.DMA)

    body = functools.partial(_sc_body, n_table, nblocks, K, with_e)
    return pl.kernel(body, out_type=tuple(out_type), mesh=mesh,
                     scratch_types=tuple(scratch),
                     compiler_params=pltpu.CompilerParams(
                         use_tc_tiling_on_sc=False))


def _seg_sum(t3, ei3, n_acc, nblocks, K, ea3=None):
    n = t3.shape[1]
    tstk = t3.reshape(NC * n, DH)
    f = _make_seg_sum(n, n_acc, nblocks, K, ea3 is not None)
    rpt = n_acc // NS
    zrows = jnp.zeros((rpt, DH), jnp.float32)
    if ea3 is not None:
        zelem = jnp.zeros((rpt,), jnp.float32)
        gp, se = f(tstk, ei3, ea3, zrows, zelem)
        return gp, se.reshape(NC, n_acc)
    return f(tstk, ei3, zrows)


def _split_store(m_ref, m):
    m_ref[0] = m[:, :DH]
    m_ref[1] = m[:, DH:]


def _tc_a_body(attr_ref, wn_ref, ws_ref, b_ref, m_ref, pre_ref):
    attr = attr_ref[...]
    _split_store(m_ref, jax.lax.dot_general(
        attr, wn_ref[...], (((1,), (0,)), ((), ())),
        preferred_element_type=jnp.float32))
    pre_ref[...] = jax.lax.dot_general(
        attr, ws_ref[...], (((1,), (0,)), ((), ())),
        preferred_element_type=jnp.float32) + b_ref[...]


def _tc_a(attr, Wn, Ws, b, blk):
    n, f = attr.shape
    grid = n // blk
    return pl.pallas_call(
        _tc_a_body,
        grid=(grid,),
        in_specs=[
            pl.BlockSpec((blk, f), lambda i: (i, 0)),
            pl.BlockSpec(Wn.shape, lambda i: (0, 0)),
            pl.BlockSpec(Ws.shape, lambda i: (0, 0)),
            pl.BlockSpec((1, D), lambda i: (0, 0)),
        ],
        out_specs=[
            pl.BlockSpec((NC, blk, DH), lambda i: (0, i, 0)),
            pl.BlockSpec((blk, D), lambda i: (i, 0)),
        ],
        out_shape=[
            jax.ShapeDtypeStruct((NC, n, DH), jnp.float32),
            jax.ShapeDtypeStruct((n, D), jnp.float32),
        ],
    )(attr, Wn, Ws, b.reshape(1, D))


def _tc_b_body(pre_ref, g_ref, s_ref, we1_ref, we2_ref, wn_ref, ws_ref,
               b2_ref, m2_ref, pre2_ref):
    g = jnp.concatenate([g_ref[0], g_ref[1]], axis=-1)
    s = s_ref[...]
    h1 = jax.nn.relu(pre_ref[...] + g + s * we1_ref[...])
    _split_store(m2_ref, jax.lax.dot_general(
        h1, wn_ref[...], (((1,), (0,)), ((), ())),
        preferred_element_type=jnp.float32))
    pre2_ref[...] = jax.lax.dot_general(
        h1, ws_ref[...], (((1,), (0,)), ((), ())),
        preferred_element_type=jnp.float32) + s * we2_ref[...] + b2_ref[...]


def _tc_b(pre1, g1p, sp, We1, We2, Wn2, Ws2, b2, blk):
    n = pre1.shape[0]
    grid = n // blk
    return pl.pallas_call(
        _tc_b_body,
        grid=(grid,),
        in_specs=[
            pl.BlockSpec((blk, D), lambda i: (i, 0)),
            pl.BlockSpec((NC, blk, DH), lambda i: (0, i, 0)),
            pl.BlockSpec((blk, 1), lambda i: (i, 0)),
            pl.BlockSpec((1, D), lambda i: (0, 0)),
            pl.BlockSpec((1, D), lambda i: (0, 0)),
            pl.BlockSpec((D, D), lambda i: (0, 0)),
            pl.BlockSpec((D, D), lambda i: (0, 0)),
            pl.BlockSpec((1, D), lambda i: (0, 0)),
        ],
        out_specs=[
            pl.BlockSpec((NC, blk, DH), lambda i: (0, i, 0)),
            pl.BlockSpec((blk, D), lambda i: (i, 0)),
        ],
        out_shape=[
            jax.ShapeDtypeStruct((NC, n, DH), jnp.float32),
            jax.ShapeDtypeStruct((n, D), jnp.float32),
        ],
    )(pre1, g1p, sp, We1.reshape(1, D), We2.reshape(1, D), Wn2,
      Ws2, b2.reshape(1, D))


def _tc_c_body(pre2_ref, g_ref, wo_ref, bo_ref, hr_ref):
    h2 = jax.nn.relu(pre2_ref[...]
                     + jnp.concatenate([g_ref[0], g_ref[1]], axis=-1))
    hr_ref[...] = jax.lax.dot_general(
        h2, wo_ref[...], (((1,), (0,)), ((), ())),
        preferred_element_type=jnp.float32) + bo_ref[...]


def _tc_c(pre2, g2p, Wo, bo, blk):
    n = pre2.shape[0]
    grid = n // blk
    return pl.pallas_call(
        _tc_c_body,
        grid=(grid,),
        in_specs=[
            pl.BlockSpec((blk, D), lambda i: (i, 0)),
            pl.BlockSpec((NC, blk, DH), lambda i: (0, i, 0)),
            pl.BlockSpec((D, D), lambda i: (0, 0)),
            pl.BlockSpec((1, D), lambda i: (0, 0)),
        ],
        out_specs=pl.BlockSpec((blk, D), lambda i: (i, 0)),
        out_shape=jax.ShapeDtypeStruct((n, D), jnp.float32),
    )(pre2, g2p, Wo, bo.reshape(1, D))


def _mm(a, b):
    return jax.lax.dot_general(a, b, (((1,), (0,)), ((), ())),
                               preferred_element_type=jnp.float32)


def _split_store_at(t_ref, lo, hi, m):
    t_ref[0, lo:hi] = m[:, :DH]
    t_ref[1, lo:hi] = m[:, DH:]


def _lig_a_body(attr_ref, eattr_ref, wn_ref, ws_ref, b_ref, we1_ref,
                we2_ref, t1_ref, pre_ref, e2_ref):
    attr = attr_ref[...]
    ea = eattr_ref[...]
    n = attr.shape[0]
    _split_store_at(t1_ref, 0, n, _mm(attr, wn_ref[...]))
    _split_store_at(t1_ref, n, t1_ref.shape[1], _mm(ea, we1_ref[...]))
    pre_ref[...] = _mm(attr, ws_ref[...]) + b_ref[...]
    e2_ref[...] = _mm(ea, we2_ref[...])


def _lig_a(attr, eattr, Wn, Ws, b, We1, We2):
    n = attr.shape[0]
    ne = eattr.shape[0]
    return pl.pallas_call(
        _lig_a_body,
        out_shape=[
            jax.ShapeDtypeStruct((NC, n + ne, DH), jnp.float32),
            jax.ShapeDtypeStruct((n, D), jnp.float32),
            jax.ShapeDtypeStruct((ne, D), jnp.float32),
        ],
    )(attr, eattr, Wn, Ws, b.reshape(1, D), We1, We2)


def _lig_b_body(pre_ref, g_ref, e2_ref, wn_ref, ws_ref, b2_ref, t2_ref,
                pre2_ref):
    n = pre_ref.shape[0]
    g = jnp.concatenate([g_ref[0, :n], g_ref[1, :n]], axis=-1)
    h1 = jax.nn.relu(pre_ref[...] + g)
    _split_store_at(t2_ref, 0, n, _mm(h1, wn_ref[...]))
    _split_store_at(t2_ref, n, t2_ref.shape[1], e2_ref[...])
    pre2_ref[...] = _mm(h1, ws_ref[...]) + b2_ref[...]


def _lig_b(pre1, g1p, e2, Wn2, Ws2, b2):
    n = pre1.shape[0]
    ne = e2.shape[0]
    return pl.pallas_call(
        _lig_b_body,
        out_shape=[
            jax.ShapeDtypeStruct((NC, n + ne, DH), jnp.float32),
            jax.ShapeDtypeStruct((n, D), jnp.float32),
        ],
    )(pre1, g1p, e2, Wn2, Ws2, b2.reshape(1, D))


def _final_body(pre2l_ref, gl_ref, wo_ref, bo_ref, lab_ref, phiw_ref,
                phib_ref, hr_ref, rx_ref, a_ref, y_ref):
    n = pre2l_ref.shape[0]
    gl = jnp.concatenate([gl_ref[0, :n], gl_ref[1, :n]], axis=-1)
    h2 = jax.nn.relu(pre2l_ref[...] + gl)
    hs_lig = _mm(h2, wo_ref[...]) + bo_ref[...]
    hl8 = _mm(lab_ref[...], hs_lig)
    hl = jax.nn.relu(_mm(hl8, phiw_ref[...]) + phib_ref[...])
    hr = hr_ref[...]
    dots = jax.lax.dot_general(hl, hr, (((1,), (1,)), ((), ())),
                               preferred_element_type=jnp.float32)
    m = jnp.max(dots, axis=1, keepdims=True)
    e = jnp.exp(dots - m)
    s = jnp.sum(e, axis=1, keepdims=True)
    a = e / s
    a_ref[...] = a
    y_ref[...] = jax.lax.dot_general(a, rx_ref[...], (((1,), (0,)), ((), ())),
                                     preferred_element_type=jnp.float32)


def _final(pre2l, g2pl, Wo, bo, labelidx, phi_W, phi_b, h_r, rec_x):
    k = labelidx.shape[0]
    n = h_r.shape[0]
    return pl.pallas_call(
        _final_body,
        out_shape=(
            jax.ShapeDtypeStruct((k, n), jnp.float32),
            jax.ShapeDtypeStruct((k, rec_x.shape[1]), jnp.float32),
        ),
    )(pre2l, g2pl, Wo, bo.reshape(1, D), labelidx, phi_W,
      phi_b.reshape(1, D), h_r, rec_x)


def _pad_edges(ei, n_table, n_nodes, n_acc, e_pad, eattr=None):
    e = ei.shape[1]
    npad = e_pad - e
    ar = jnp.arange(npad, dtype=jnp.int32)
    fill = jnp.stack([ar % n_table, n_nodes + ar % (n_acc - n_nodes)])
    ei3 = jnp.concatenate([ei, fill], axis=1).reshape(2, -1, CH)
    if eattr is None:
        return ei3
    ea3 = jnp.concatenate(
        [eattr, jnp.zeros((npad, 1), jnp.float32)]).reshape(-1, CH)
    return ei3, ea3


REC_K = 8
REC_NBLOCKS = 98
REC_EPAD = NS * REC_NBLOCKS * REC_K * CH
REC_NACC = 50432

LIG_K = 8
LIG_NBLOCKS = 2
LIG_EPAD = NS * LIG_NBLOCKS * LIG_K * CH
LIG_NACC = 1024


def kernel(rec_attr, rec_x, rec_edge_index, rec_edge_attr, lig_attr,
           lig_edge_index, lig_edge_attr, labelidx, Wr_self1, Wr_nbr1, Wr_e1,
           br1, Wr_self2, Wr_nbr2, Wr_e2, br2, Wr_out, br_out, Wl_self1,
           Wl_nbr1, Wl_e1, bl1, Wl_self2, Wl_nbr2, Wl_e2, bl2, Wl_out,
           bl_out, phi_W, phi_b, Ascaler1, Ascaler2):
    n_rec = rec_attr.shape[0]

    ei3, ea3 = _pad_edges(rec_edge_index, n_rec, n_rec, REC_NACC, REC_EPAD,
                          rec_edge_attr)
    m1, pre1 = _tc_a(rec_attr, Wr_nbr1, Wr_self1, br1, blk=5000)
    g1p, sep = _seg_sum(m1, ei3, REC_NACC, REC_NBLOCKS, REC_K, ea3=ea3)
    sp = sep[0][:, None]
    m2, pre2 = _tc_b(pre1, g1p, sp, Wr_e1[0], Wr_e2[0], Wr_nbr2, Wr_self2,
                     br2, blk=5000)
    (g2p,) = _seg_sum(m2, ei3, REC_NACC, REC_NBLOCKS, REC_K)
    hs_rec = _tc_c(pre2, g2p, Wr_out, br_out, blk=5000)

    n_lig = lig_attr.shape[0]
    e_lig = lig_edge_attr.shape[0]
    virt = n_lig + jnp.arange(e_lig, dtype=jnp.int32)
    aei = jnp.stack([jnp.concatenate([lig_edge_index[0], virt]),
                     jnp.concatenate([lig_edge_index[1],
                                      lig_edge_index[1]])])
    lei3 = _pad_edges(aei, n_lig + e_lig, n_lig, LIG_NACC, LIG_EPAD)
    lt1, lpre1, le2 = _lig_a(lig_attr, lig_edge_attr, Wl_nbr1, Wl_self1,
                             bl1, Wl_e1, Wl_e2)
    (lg1p,) = _seg_sum(lt1, lei3, LIG_NACC, LIG_NBLOCKS, LIG_K)
    lt2, lpre2 = _lig_b(lpre1, lg1p, le2, Wl_nbr2, Wl_self2, bl2)
    (lg2p,) = _seg_sum(lt2, lei3, LIG_NACC, LIG_NBLOCKS, LIG_K)

    A, Yrec = _final(lpre2, lg2p, Wl_out, bl_out, labelidx, phi_W, phi_b,
                     hs_rec, rec_x)
    return (Yrec[None], A)


# speedup vs baseline: 19.6606x; 1.3235x over previous
import functools

import jax
import jax.numpy as jnp
from jax import lax
from jax.experimental import pallas as pl
from jax.experimental.pallas import tpu as pltpu
from jax.experimental.pallas import tpu_sc as plsc

NC = 2
NS = 16
NW = NC * NS
CH = 128
D = 32
DH = D // NC


def _sc_body(n_table, nblocks, K, with_e, *refs):
    if with_e:
        (table, ei3, ea3, zrows, zelem, out_rows, out_e,
         sbuf, dbuf, ebuf, rows, ebounce, acc, acce,
         gsem, ssem, isem, esem) = refs
    else:
        (table, ei3, zrows, out_rows,
         sbuf, dbuf, rows, acc, gsem, ssem, isem) = refs
    cid = lax.axis_index("c")
    sid = lax.axis_index("s")
    n_acc = acc.shape[0]
    rpt = n_acc // NS
    pltpu.sync_copy(zrows, acc.at[pl.ds(sid * rpt, rpt)])
    if with_e:
        pltpu.sync_copy(zelem, ebounce)
        pltpu.sync_copy(ebounce, acce.at[pl.ds(sid * rpt, rpt)])
    plsc.subcore_barrier()

    wbase = sid * (nblocks * K)

    off = jnp.full((16,), n_table, jnp.int32)

    def idx_issue(bn, s):
        rowbase = wbase + bn * K
        pltpu.async_copy(ei3.at[0, pl.ds(rowbase, K)], sbuf.at[s], isem)
        pltpu.async_copy(ei3.at[1, pl.ds(rowbase, K)], dbuf.at[s], isem)
        if with_e:
            pltpu.async_copy(ea3.at[pl.ds(rowbase, K)], ebuf.at[s], isem)

    def idx_wait(bn, s):
        rowbase = wbase + bn * K
        pltpu.make_async_copy(ei3.at[0, pl.ds(rowbase, K)], sbuf.at[s],
                              isem).wait()
        pltpu.make_async_copy(ei3.at[1, pl.ds(rowbase, K)], dbuf.at[s],
                              isem).wait()
        if with_e:
            pltpu.make_async_copy(ea3.at[pl.ds(rowbase, K)], ebuf.at[s],
                                  isem).wait()

    def fire(s):
        @pl.when(cid == 1)
        def _():
            for j in range(K):
                for i in range(CH // 16):
                    sl = pl.ds(i * 16, 16)
                    sbuf[s, j, sl] = sbuf[s, j, sl] + off
        for j in range(K):
            pltpu.async_copy(table.at[sbuf.at[s, j]], rows.at[s, j], gsem)

    def drain(s):
        for j in range(K):
            pltpu.make_async_copy(rows.at[s, j], acc.at[dbuf.at[s, j]],
                                  ssem).wait()
            if with_e:
                pltpu.make_async_copy(ebuf.at[s, j], acce.at[dbuf.at[s, j]],
                                      esem).wait()

    def process(b, s):
        s_prev = (s + 2) % 3
        s_next = (s + 1) % 3
        for j in range(K):
            pltpu.make_async_copy(table.at[sbuf.at[s, j]], rows.at[s, j],
                                  gsem).wait()
            pltpu.async_copy(rows.at[s, j], acc.at[dbuf.at[s, j]], ssem,
                             add=True)
            if with_e:
                pltpu.async_copy(ebuf.at[s, j], acce.at[dbuf.at[s, j]],
                                 esem, add=True)
        @pl.when(b > 0)
        def _():
            drain(s_prev)

        @pl.when(b + 2 < nblocks)
        def _():
            idx_issue(b + 2, s_prev)

        @pl.when(b + 1 < nblocks)
        def _():
            idx_wait(b + 1, s_next)
            fire(s_next)

    idx_issue(0, 0)
    idx_wait(0, 0)
    fire(0)
    idx_issue(1, 1)

    def blk3(i, carry):
        process(3 * i, 0)
        process(3 * i + 1, 1)
        process(3 * i + 2, 2)
        return carry

    lax.fori_loop(0, nblocks // 3, blk3, 0)
    drain((nblocks - 1) % 3)
    plsc.subcore_barrier()
    pltpu.sync_copy(acc.at[pl.ds(sid * rpt, rpt)],
                    out_rows.at[cid, pl.ds(sid * rpt, rpt)])
    if with_e:
        pltpu.sync_copy(acce.at[pl.ds(sid * rpt, rpt)], ebounce)
        pltpu.sync_copy(ebounce,
                        out_e.at[pl.ds(cid * n_acc + sid * rpt, rpt)])


def _make_seg_sum(n_table, n_acc, nblocks, K, with_e):
    mesh = plsc.VectorSubcoreMesh(core_axis_name="c", subcore_axis_name="s")
    rpt = n_acc // NS
    out_type = [jax.ShapeDtypeStruct((NC, n_acc, DH), jnp.float32)]
    scratch = [
        pltpu.VMEM((3, K, CH), jnp.int32),
        pltpu.VMEM((3, K, CH), jnp.int32),
    ]
    if with_e:
        out_type.append(jax.ShapeDtypeStruct((NC * n_acc,), jnp.float32))
        scratch.append(pltpu.VMEM((3, K, CH), jnp.float32))
    scratch.append(pltpu.VMEM((3, K, CH, DH), jnp.float32))
    if with_e:
        scratch.append(pltpu.VMEM((rpt,), jnp.float32))
    scratch.append(pltpu.VMEM_SHARED((n_acc, DH), jnp.float32))
    if with_e:
        scratch.append(pltpu.VMEM_SHARED((n_acc,), jnp.float32))
    scratch.append(pltpu.SemaphoreType.DMA)
    scratch.append(pltpu.SemaphoreType.DMA)
    scratch.append(pltpu.SemaphoreType.DMA)
    if with_e:
        scratch.append(pltpu.SemaphoreType.DMA)

    body = functools.partial(_sc_body, n_table, nblocks, K, with_e)
    return pl.kernel(body, out_type=tuple(out_type), mesh=mesh,
                     scratch_types=tuple(scratch),
                     compiler_params=pltpu.CompilerParams(
                         use_tc_tiling_on_sc=False))


def _seg_sum(t3, ei3, n_acc, nblocks, K, ea3=None):
    n = t3.shape[1]
    tstk = t3.reshape(NC * n, DH)
    f = _make_seg_sum(n, n_acc, nblocks, K, ea3 is not None)
    rpt = n_acc // NS
    zrows = jnp.zeros((rpt, DH), jnp.float32)
    if ea3 is not None:
        zelem = jnp.zeros((rpt,), jnp.float32)
        gp, se = f(tstk, ei3, ea3, zrows, zelem)
        return gp, se.reshape(NC, n_acc)
    return f(tstk, ei3, zrows)


def _split_store(m_ref, m):
    m_ref[0] = m[:, :DH]
    m_ref[1] = m[:, DH:]


def _tc_a_body(attr_ref, wn_ref, ws_ref, b_ref, m_ref, pre_ref):
    attr = attr_ref[...]
    _split_store(m_ref, jax.lax.dot_general(
        attr, wn_ref[...], (((1,), (0,)), ((), ())),
        preferred_element_type=jnp.float32))
    pre_ref[...] = jax.lax.dot_general(
        attr, ws_ref[...], (((1,), (0,)), ((), ())),
        preferred_element_type=jnp.float32) + b_ref[...]


def _tc_a(attr, Wn, Ws, b, blk):
    n, f = attr.shape
    grid = n // blk
    return pl.pallas_call(
        _tc_a_body,
        grid=(grid,),
        in_specs=[
            pl.BlockSpec((blk, f), lambda i: (i, 0)),
            pl.BlockSpec(Wn.shape, lambda i: (0, 0)),
            pl.BlockSpec(Ws.shape, lambda i: (0, 0)),
            pl.BlockSpec((1, D), lambda i: (0, 0)),
        ],
        out_specs=[
            pl.BlockSpec((NC, blk, DH), lambda i: (0, i, 0)),
            pl.BlockSpec((blk, D), lambda i: (i, 0)),
        ],
        out_shape=[
            jax.ShapeDtypeStruct((NC, n, DH), jnp.float32),
            jax.ShapeDtypeStruct((n, D), jnp.float32),
        ],
    )(attr, Wn, Ws, b.reshape(1, D))


def _tc_b_body(pre_ref, g_ref, s_ref, we1_ref, we2_ref, wn_ref, ws_ref,
               b2_ref, m2_ref, pre2_ref):
    g = jnp.concatenate([g_ref[0], g_ref[1]], axis=-1)
    s = s_ref[...]
    h1 = jax.nn.relu(pre_ref[...] + g + s * we1_ref[...])
    _split_store(m2_ref, jax.lax.dot_general(
        h1, wn_ref[...], (((1,), (0,)), ((), ())),
        preferred_element_type=jnp.float32))
    pre2_ref[...] = jax.lax.dot_general(
        h1, ws_ref[...], (((1,), (0,)), ((), ())),
        preferred_element_type=jnp.float32) + s * we2_ref[...] + b2_ref[...]


def _tc_b(pre1, g1p, sp, We1, We2, Wn2, Ws2, b2, blk):
    n = pre1.shape[0]
    grid = n // blk
    return pl.pallas_call(
        _tc_b_body,
        grid=(grid,),
        in_specs=[
            pl.BlockSpec((blk, D), lambda i: (i, 0)),
            pl.BlockSpec((NC, blk, DH), lambda i: (0, i, 0)),
            pl.BlockSpec((blk, 1), lambda i: (i, 0)),
            pl.BlockSpec((1, D), lambda i: (0, 0)),
            pl.BlockSpec((1, D), lambda i: (0, 0)),
            pl.BlockSpec((D, D), lambda i: (0, 0)),
            pl.BlockSpec((D, D), lambda i: (0, 0)),
            pl.BlockSpec((1, D), lambda i: (0, 0)),
        ],
        out_specs=[
            pl.BlockSpec((NC, blk, DH), lambda i: (0, i, 0)),
            pl.BlockSpec((blk, D), lambda i: (i, 0)),
        ],
        out_shape=[
            jax.ShapeDtypeStruct((NC, n, DH), jnp.float32),
            jax.ShapeDtypeStruct((n, D), jnp.float32),
        ],
    )(pre1, g1p, sp, We1.reshape(1, D), We2.reshape(1, D), Wn2,
      Ws2, b2.reshape(1, D))


def _tc_c_body(pre2_ref, g_ref, wo_ref, bo_ref, hr_ref):
    h2 = jax.nn.relu(pre2_ref[...]
                     + jnp.concatenate([g_ref[0], g_ref[1]], axis=-1))
    hr_ref[...] = jax.lax.dot_general(
        h2, wo_ref[...], (((1,), (0,)), ((), ())),
        preferred_element_type=jnp.float32) + bo_ref[...]


def _tc_c(pre2, g2p, Wo, bo, blk):
    n = pre2.shape[0]
    grid = n // blk
    return pl.pallas_call(
        _tc_c_body,
        grid=(grid,),
        in_specs=[
            pl.BlockSpec((blk, D), lambda i: (i, 0)),
            pl.BlockSpec((NC, blk, DH), lambda i: (0, i, 0)),
            pl.BlockSpec((D, D), lambda i: (0, 0)),
            pl.BlockSpec((1, D), lambda i: (0, 0)),
        ],
        out_specs=pl.BlockSpec((blk, D), lambda i: (i, 0)),
        out_shape=jax.ShapeDtypeStruct((n, D), jnp.float32),
    )(pre2, g2p, Wo, bo.reshape(1, D))


def _mm(a, b):
    return jax.lax.dot_general(a, b, (((1,), (0,)), ((), ())),
                               preferred_element_type=jnp.float32)


def _split_store_at(t_ref, lo, hi, m):
    t_ref[0, lo:hi] = m[:, :DH]
    t_ref[1, lo:hi] = m[:, DH:]


def _lig_a_body(attr_ref, eattr_ref, wn_ref, ws_ref, b_ref, we1_ref,
                we2_ref, t1_ref, pre_ref, e2_ref):
    attr = attr_ref[...]
    ea = eattr_ref[...]
    n = attr.shape[0]
    _split_store_at(t1_ref, 0, n, _mm(attr, wn_ref[...]))
    _split_store_at(t1_ref, n, t1_ref.shape[1], _mm(ea, we1_ref[...]))
    pre_ref[...] = _mm(attr, ws_ref[...]) + b_ref[...]
    e2_ref[...] = _mm(ea, we2_ref[...])


def _lig_a(attr, eattr, Wn, Ws, b, We1, We2):
    n = attr.shape[0]
    ne = eattr.shape[0]
    return pl.pallas_call(
        _lig_a_body,
        out_shape=[
            jax.ShapeDtypeStruct((NC, n + ne, DH), jnp.float32),
            jax.ShapeDtypeStruct((n, D), jnp.float32),
            jax.ShapeDtypeStruct((ne, D), jnp.float32),
        ],
    )(attr, eattr, Wn, Ws, b.reshape(1, D), We1, We2)


def _lig_b_body(pre_ref, g_ref, e2_ref, wn_ref, ws_ref, b2_ref, t2_ref,
                pre2_ref):
    n = pre_ref.shape[0]
    g = jnp.concatenate([g_ref[0, :n], g_ref[1, :n]], axis=-1)
    h1 = jax.nn.relu(pre_ref[...] + g)
    _split_store_at(t2_ref, 0, n, _mm(h1, wn_ref[...]))
    _split_store_at(t2_ref, n, t2_ref.shape[1], e2_ref[...])
    pre2_ref[...] = _mm(h1, ws_ref[...]) + b2_ref[...]


def _lig_b(pre1, g1p, e2, Wn2, Ws2, b2):
    n = pre1.shape[0]
    ne = e2.shape[0]
    return pl.pallas_call(
        _lig_b_body,
        out_shape=[
            jax.ShapeDtypeStruct((NC, n + ne, DH), jnp.float32),
            jax.ShapeDtypeStruct((n, D), jnp.float32),
        ],
    )(pre1, g1p, e2, Wn2, Ws2, b2.reshape(1, D))


def _final_body(pre2l_ref, gl_ref, wo_ref, bo_ref, lab_ref, phiw_ref,
                phib_ref, hr_ref, rx_ref, a_ref, y_ref):
    n = pre2l_ref.shape[0]
    gl = jnp.concatenate([gl_ref[0, :n], gl_ref[1, :n]], axis=-1)
    h2 = jax.nn.relu(pre2l_ref[...] + gl)
    hs_lig = _mm(h2, wo_ref[...]) + bo_ref[...]
    hl8 = _mm(lab_ref[...], hs_lig)
    hl = jax.nn.relu(_mm(hl8, phiw_ref[...]) + phib_ref[...])
    hr = hr_ref[...]
    dots = jax.lax.dot_general(hl, hr, (((1,), (1,)), ((), ())),
                               preferred_element_type=jnp.float32)
    m = jnp.max(dots, axis=1, keepdims=True)
    e = jnp.exp(dots - m)
    s = jnp.sum(e, axis=1, keepdims=True)
    a = e / s
    a_ref[...] = a
    y_ref[...] = jax.lax.dot_general(a, rx_ref[...], (((1,), (0,)), ((), ())),
                                     preferred_element_type=jnp.float32)


def _final(pre2l, g2pl, Wo, bo, labelidx, phi_W, phi_b, h_r, rec_x):
    k = labelidx.shape[0]
    n = h_r.shape[0]
    return pl.pallas_call(
        _final_body,
        out_shape=(
            jax.ShapeDtypeStruct((k, n), jnp.float32),
            jax.ShapeDtypeStruct((k, rec_x.shape[1]), jnp.float32),
        ),
    )(pre2l, g2pl, Wo, bo.reshape(1, D), labelidx, phi_W,
      phi_b.reshape(1, D), h_r, rec_x)


def _pad_edges(ei, n_table, n_nodes, n_acc, e_pad, eattr=None):
    e = ei.shape[1]
    npad = e_pad - e
    ar = jnp.arange(npad, dtype=jnp.int32)
    fill = jnp.stack([ar % n_table, n_nodes + ar % (n_acc - n_nodes)])
    ei3 = jnp.concatenate([ei, fill], axis=1).reshape(2, -1, CH)
    if eattr is None:
        return ei3
    ea3 = jnp.concatenate(
        [eattr, jnp.zeros((npad, 1), jnp.float32)]).reshape(-1, CH)
    return ei3, ea3


REC_K = 8
REC_NBLOCKS = 99
REC_EPAD = NS * REC_NBLOCKS * REC_K * CH
REC_NACC = 50432

LIG_K = 8
LIG_NBLOCKS = 3
LIG_EPAD = NS * LIG_NBLOCKS * LIG_K * CH
LIG_NACC = 2048


def kernel(rec_attr, rec_x, rec_edge_index, rec_edge_attr, lig_attr,
           lig_edge_index, lig_edge_attr, labelidx, Wr_self1, Wr_nbr1, Wr_e1,
           br1, Wr_self2, Wr_nbr2, Wr_e2, br2, Wr_out, br_out, Wl_self1,
           Wl_nbr1, Wl_e1, bl1, Wl_self2, Wl_nbr2, Wl_e2, bl2, Wl_out,
           bl_out, phi_W, phi_b, Ascaler1, Ascaler2):
    n_rec = rec_attr.shape[0]

    ei3, ea3 = _pad_edges(rec_edge_index, n_rec, n_rec, REC_NACC, REC_EPAD,
                          rec_edge_attr)
    m1, pre1 = _tc_a(rec_attr, Wr_nbr1, Wr_self1, br1, blk=5000)
    g1p, sep = _seg_sum(m1, ei3, REC_NACC, REC_NBLOCKS, REC_K, ea3=ea3)
    sp = sep[0][:, None]
    m2, pre2 = _tc_b(pre1, g1p, sp, Wr_e1[0], Wr_e2[0], Wr_nbr2, Wr_self2,
                     br2, blk=5000)
    (g2p,) = _seg_sum(m2, ei3, REC_NACC, REC_NBLOCKS, REC_K)
    hs_rec = _tc_c(pre2, g2p, Wr_out, br_out, blk=5000)

    n_lig = lig_attr.shape[0]
    e_lig = lig_edge_attr.shape[0]
    virt = n_lig + jnp.arange(e_lig, dtype=jnp.int32)
    aei = jnp.stack([jnp.concatenate([lig_edge_index[0], virt]),
                     jnp.concatenate([lig_edge_index[1],
                                      lig_edge_index[1]])])
    lei3 = _pad_edges(aei, n_lig + e_lig, n_lig, LIG_NACC, LIG_EPAD)
    lt1, lpre1, le2 = _lig_a(lig_attr, lig_edge_attr, Wl_nbr1, Wl_self1,
                             bl1, Wl_e1, Wl_e2)
    (lg1p,) = _seg_sum(lt1, lei3, LIG_NACC, LIG_NBLOCKS, LIG_K)
    lt2, lpre2 = _lig_b(lpre1, lg1p, le2, Wl_nbr2, Wl_self2, bl2)
    (lg2p,) = _seg_sum(lt2, lei3, LIG_NACC, LIG_NBLOCKS, LIG_K)

    A, Yrec = _final(lpre2, lg2p, Wl_out, bl_out, labelidx, phi_W, phi_b,
                     hs_rec, rec_x)
    return (Yrec[None], A)
